# baseline (speedup 1.0000x reference)
import functools

import jax
import jax.numpy as jnp
from jax import lax
from jax.experimental import pallas as pl
from jax.experimental.pallas import tpu as pltpu
from jax.experimental.pallas import tpu_sc as plsc

_NC = 2
_NS = 16
_NW = _NC * _NS

_IW = 125


def _sc_gather_pair(table, idx_a2d, idx_b2d):
    n_rows = idx_a2d.shape[0]
    E = n_rows * _IW
    per_w = E // _NW
    CH = 8 * _IW
    iters = per_w // CH
    mesh = plsc.VectorSubcoreMesh(core_axis_name="c", subcore_axis_name="s")

    @functools.partial(
        pl.kernel,
        out_type=(
            jax.ShapeDtypeStruct((E, 4), jnp.float32),
            jax.ShapeDtypeStruct((E, 4), jnp.float32),
        ),
        mesh=mesh,
        scratch_types=[
            pltpu.VMEM((8, _IW), jnp.int32),
            pltpu.VMEM((8, _IW), jnp.int32),
            pltpu.VMEM((CH, 4), jnp.float32),
            pltpu.VMEM((CH, 4), jnp.float32),
            pltpu.SemaphoreType.DMA,
            pltpu.SemaphoreType.DMA,
        ],
        compiler_params=pltpu.CompilerParams(use_tc_tiling_on_sc=False),
    )
    def k(x_hbm, ia_hbm, ib_hbm, oa_hbm, ob_hbm, ia_v, ib_v, ra_v, rb_v,
          sem_a, sem_b):
        c = lax.axis_index("c")
        s = lax.axis_index("s")
        wid = s * _NC + c
        base = wid * per_w

        def body(it, carry):
            eoff = pl.multiple_of(base + it * CH, CH)
            roff = pl.multiple_of(eoff // _IW, 8)
            pltpu.sync_copy(ia_hbm.at[pl.ds(roff, 8), :], ia_v)
            pltpu.sync_copy(ib_hbm.at[pl.ds(roff, 8), :], ib_v)
            cps = []
            for j in range(8):
                cps.append(pltpu.async_copy(
                    x_hbm.at[ia_v.at[j]],
                    ra_v.at[pl.ds(j * _IW, _IW), :], sem_a))
                cps.append(pltpu.async_copy(
                    x_hbm.at[ib_v.at[j]],
                    rb_v.at[pl.ds(j * _IW, _IW), :], sem_b))
            for cp in cps:
                cp.wait()
            pltpu.sync_copy(ra_v, oa_hbm.at[pl.ds(eoff, CH), :])
            pltpu.sync_copy(rb_v, ob_hbm.at[pl.ds(eoff, CH), :])
            return carry

        lax.fori_loop(0, iters, body, 0)

    return k(table, idx_a2d, idx_b2d)


def _sc_segment_sum(dst2d, msg, zeros_n4):
    N = zeros_n4.shape[0]
    n_rows = dst2d.shape[0]
    E = n_rows * _IW
    per_sc = E // _NC
    per_tile = per_sc // _NS
    CH = 16 * _IW
    iters = per_tile // CH
    per_tile_n = N // _NS
    mesh = plsc.VectorSubcoreMesh(core_axis_name="c", subcore_axis_name="s")

    @functools.partial(
        pl.kernel,
        out_type=jax.ShapeDtypeStruct((_NC, N, 4), jnp.float32),
        mesh=mesh,
        scratch_types=[
            pltpu.VMEM((16, _IW), jnp.int32),
            pltpu.VMEM((CH, 4), jnp.float32),
            pltpu.VMEM_SHARED((N, 4), jnp.float32),
        ],
        compiler_params=pltpu.CompilerParams(use_tc_tiling_on_sc=False),
    )
    def k(d_hbm, m_hbm, z_hbm, out_hbm, idx_v, upd_v, acc_sh):
        c = lax.axis_index("c")
        s = lax.axis_index("s")
        noff = s * per_tile_n
        pltpu.sync_copy(z_hbm.at[pl.ds(noff, per_tile_n), :],
                        acc_sh.at[pl.ds(noff, per_tile_n), :])
        plsc.subcore_barrier()
        base = c * per_sc + s * per_tile

        def body(it, carry):
            eoff = pl.multiple_of(base + it * CH, CH)
            roff = pl.multiple_of(eoff // _IW, 16)
            pltpu.sync_copy(d_hbm.at[pl.ds(roff, 16), :], idx_v)
            pltpu.sync_copy(m_hbm.at[pl.ds(eoff, CH), :], upd_v)
            for j in range(16):
                pltpu.sync_copy(upd_v.at[pl.ds(j * _IW, _IW), :],
                                acc_sh.at[idx_v.at[j]], add=True)
            return carry

        lax.fori_loop(0, iters, body, 0)
        plsc.subcore_barrier()
        pltpu.sync_copy(acc_sh.at[pl.ds(noff, per_tile_n), :],
                        out_hbm.at[c, pl.ds(noff, per_tile_n), :])

    return k(dst2d, msg, zeros_n4)


def _split_bf16(a):
    hi = a.astype(jnp.bfloat16)
    lo = (a - hi.astype(jnp.float32)).astype(jnp.bfloat16)
    return hi, lo


def _dot3(m, w_hi, w_lo):
    m_hi, m_lo = _split_bf16(m)
    acc = jnp.dot(m_hi, w_hi, preferred_element_type=jnp.float32)
    acc += jnp.dot(m_hi, w_lo, preferred_element_type=jnp.float32)
    acc += jnp.dot(m_lo, w_hi, preferred_element_type=jnp.float32)
    return acc


def _edge_mlp_body(gi, gj, ea, w1h, w1l, b1, w2h, w2l, b2, w3h, w3l, b3,
                   out, *, sigmoid):
    m = jnp.concatenate([gi[...], gj[...], ea[...]], axis=1)
    h = jnp.maximum(_dot3(m, w1h[...], w1l[...]) + b1[...], 0.0)
    h = jnp.maximum(_dot3(h, w2h[...], w2l[...]) + b2[...], 0.0)
    o = _dot3(h, w3h[...], w3l[...]) + b3[...]
    if sigmoid:
        o = jax.nn.sigmoid(o)
    out[...] = o


def _edge_mlp(gi, gj, ea, W1cat, b1, W2, b2, W3, b3, *, sigmoid, bm=4096):
    E = gi.shape[0]
    n_out = W3.shape[1]
    w1h, w1l = _split_bf16(W1cat)
    w2h, w2l = _split_bf16(W2)
    w3h, w3l = _split_bf16(W3)
    eb = lambda n: pl.BlockSpec((bm, n), lambda i: (i, 0))
    wb = lambda a: pl.BlockSpec(a.shape, lambda i: (0, 0))
    return pl.pallas_call(
        functools.partial(_edge_mlp_body, sigmoid=sigmoid),
        grid=(E // bm,),
        in_specs=[eb(4), eb(4), eb(4),
                  wb(w1h), wb(w1l), wb(b1),
                  wb(w2h), wb(w2l), wb(b2),
                  wb(w3h), wb(w3l), wb(b3)],
        out_specs=eb(n_out),
        out_shape=jax.ShapeDtypeStruct((E, n_out), jnp.float32),
    )(gi, gj, ea, w1h, w1l, b1, w2h, w2l, b2, w3h, w3l, b3)


def _node_mlp_body(x, p0, p1, w1, b1, w2, b2, w3, b3, out):
    m = jnp.concatenate([x[...], p0[...] + p1[...]], axis=1)
    h = jnp.dot(m, w1[...], preferred_element_type=jnp.float32) + b1[...]
    h = jnp.maximum(h, 0.0)
    h = jnp.dot(h, w2[...], preferred_element_type=jnp.float32) + b2[...]
    h = jnp.maximum(h, 0.0)
    out[...] = jnp.dot(h, w3[...], preferred_element_type=jnp.float32) + b3[...]


def _node_mlp(x, p0, p1, w1, b1, w2, b2, w3, b3, *, bn=2000):
    N = x.shape[0]
    eb = lambda n: pl.BlockSpec((bn, n), lambda i: (i, 0))
    wb = lambda a: pl.BlockSpec(a.shape, lambda i: (0, 0))
    return pl.pallas_call(
        _node_mlp_body,
        grid=(N // bn,),
        in_specs=[eb(3), eb(4), eb(4),
                  wb(w1), wb(b1), wb(w2), wb(b2), wb(w3), wb(b3)],
        out_specs=eb(4),
        out_shape=jax.ShapeDtypeStruct((N, 4), jnp.float32),
    )(x, p0, p1, w1, b1, w2, b2, w3, b3)


def _cat_w1(W1):
    z = jnp.zeros((1, W1.shape[1]), W1.dtype)
    return jnp.concatenate([W1[0:3], z, W1[3:6], z, W1[6:10]], axis=0)


def kernel(x, edge_index, edge_attr,
           R1_W1, R1_b1, R1_W2, R1_b2, R1_W3, R1_b3,
           O_W1, O_b1, O_W2, O_b2, O_W3, O_b3,
           R2_W1, R2_b1, R2_W2, R2_b2, R2_W3, R2_b3):
    N = x.shape[0]
    E = edge_index.shape[1]

    xpad = jnp.concatenate([x, jnp.zeros((N, 1), x.dtype)], axis=1)
    dst2d = edge_index[1].reshape(E // _IW, _IW)
    src2d = edge_index[0].reshape(E // _IW, _IW)
    n_pad = ((N + _NS * 8 - 1) // (_NS * 8)) * (_NS * 8)
    zeros_n4 = jnp.zeros((n_pad, 4), jnp.float32)

    gi, gj = _sc_gather_pair(xpad, dst2d, src2d)
    emsg = _edge_mlp(
        gi, gj, edge_attr,
        _cat_w1(R1_W1), R1_b1.reshape(1, -1),
        R1_W2, R1_b2.reshape(1, -1),
        R1_W3, R1_b3.reshape(1, -1),
        sigmoid=False)

    parts = _sc_segment_sum(dst2d, emsg, zeros_n4)
    w3p = jnp.concatenate([O_W3, jnp.zeros((O_W3.shape[0], 1), O_W3.dtype)],
                          axis=1)
    b3p = jnp.concatenate([O_b3, jnp.zeros((1,), O_b3.dtype)])
    xt_pad = _node_mlp(
        x, parts[0, :N], parts[1, :N],
        O_W1, O_b1.reshape(1, -1),
        O_W2, O_b2.reshape(1, -1),
        w3p, b3p.reshape(1, -1))

    gi2, gj2 = _sc_gather_pair(xt_pad, dst2d, src2d)
    return _edge_mlp(
        gi2, gj2, emsg,
        _cat_w1(R2_W1), R2_b1.reshape(1, -1),
        R2_W2, R2_b2.reshape(1, -1),
        R2_W3, R2_b3.reshape(1, -1),
        sigmoid=True)


# speedup vs baseline: 1.1225x; 1.1225x over previous
import functools

import jax
import jax.numpy as jnp
from jax import lax
from jax.experimental import pallas as pl
from jax.experimental.pallas import tpu as pltpu
from jax.experimental.pallas import tpu_sc as plsc

_NC = 2
_NS = 16
_NW = _NC * _NS

_P = 5
_R = 128
_CH = _P * _R


def _sc_params():
    return pltpu.CompilerParams(use_tc_tiling_on_sc=False)


def _mesh():
    return plsc.VectorSubcoreMesh(core_axis_name="c", subcore_axis_name="s")


def _sc_gather_packed(table8, dst8, src8, ea=None):
    C = dst8.shape[0]
    E5 = C * _R
    FW = 24 if ea is not None else 16
    scratch = [
        pltpu.VMEM((8, _R), jnp.int32),
        pltpu.VMEM((8, _R), jnp.int32),
        pltpu.VMEM((_P * _R, 8), jnp.float32),
        pltpu.VMEM((_P * _R, 8), jnp.float32),
        pltpu.SemaphoreType.DMA,
        pltpu.SemaphoreType.DMA,
    ]
    if ea is not None:
        scratch.append(pltpu.VMEM((_P * _R, 8), jnp.float32))
    n_lo = C // _NW
    n_hi_workers = C - n_lo * _NW

    @functools.partial(
        pl.kernel,
        out_type=jax.ShapeDtypeStruct((E5, FW * _P), jnp.float32),
        mesh=_mesh(),
        scratch_types=scratch,
        compiler_params=_sc_params(),
    )
    def k(*refs):
        if ea is not None:
            (t_hbm, d_hbm, s_hbm, e_hbm, out_hbm,
             id_v, is_v, bd_v, bs_v, sem_d, sem_s, be_v) = refs
        else:
            (t_hbm, d_hbm, s_hbm, out_hbm,
             id_v, is_v, bd_v, bs_v, sem_d, sem_s) = refs
        c_ax = lax.axis_index("c")
        s_ax = lax.axis_index("s")
        wid = s_ax * _NC + c_ax
        iters = n_lo + jnp.where(wid < n_hi_workers, 1, 0)

        def body(it, carry):
            c = it * _NW + wid
            r0 = pl.multiple_of(c * _R, _R)
            pltpu.sync_copy(d_hbm.at[c], id_v)
            pltpu.sync_copy(s_hbm.at[c], is_v)
            cps = []
            for kk in range(_P):
                cps.append(pltpu.async_copy(
                    t_hbm.at[id_v.at[kk]],
                    bd_v.at[pl.ds(kk * _R, _R), :], sem_d))
                cps.append(pltpu.async_copy(
                    t_hbm.at[is_v.at[kk]],
                    bs_v.at[pl.ds(kk * _R, _R), :], sem_s))
            if ea is not None:
                e0 = pl.multiple_of(c * _CH, _CH)
                for kk in range(_P):
                    pltpu.sync_copy(e_hbm.at[pl.ds(e0 + kk * _R, _R), :],
                                    be_v.at[pl.ds(kk * _R, _R), :])
            for cp in cps:
                cp.wait()
            for kk in range(_P):
                pltpu.sync_copy(
                    bd_v.at[pl.ds(kk * _R, _R), :],
                    out_hbm.at[pl.ds(r0, _R), pl.ds(FW * kk, 8)])
                pltpu.sync_copy(
                    bs_v.at[pl.ds(kk * _R, _R), :],
                    out_hbm.at[pl.ds(r0, _R), pl.ds(FW * kk + 8, 8)])
            if ea is not None:
                for kk in range(_P):
                    pltpu.sync_copy(
                        be_v.at[pl.ds(kk * _R, _R), :],
                        out_hbm.at[pl.ds(r0, _R), pl.ds(FW * kk + 16, 8)])
            return carry

        lax.fori_loop(0, iters, body, 0)

    if ea is not None:
        return k(table8, dst8, src8, ea)
    return k(table8, dst8, src8)


def _sc_segment_sum(dst8, em_p, zeros_n8):
    NP = zeros_n8.shape[0]
    C = dst8.shape[0]
    per_tile_n = NP // _NS
    c_half = C // _NC

    @functools.partial(
        pl.kernel,
        out_type=jax.ShapeDtypeStruct((_NC, NP, 8), jnp.float32),
        mesh=_mesh(),
        scratch_types=[
            pltpu.VMEM((8, _R), jnp.int32),
            pltpu.VMEM((_R, 8), jnp.float32),
            pltpu.VMEM_SHARED((NP, 8), jnp.float32),
        ],
        compiler_params=_sc_params(),
    )
    def k(d_hbm, m_hbm, z_hbm, out_hbm, idx_v, upd_v, acc_sh):
        c_ax = lax.axis_index("c")
        s_ax = lax.axis_index("s")
        noff = pl.multiple_of(s_ax * per_tile_n, 8)
        pltpu.sync_copy(z_hbm.at[pl.ds(noff, per_tile_n), :],
                        acc_sh.at[pl.ds(noff, per_tile_n), :])
        plsc.subcore_barrier()
        n_lo = c_half // _NS
        n_hi = c_half - n_lo * _NS
        iters = n_lo + jnp.where(s_ax < n_hi, 1, 0)

        def body(it, carry):
            c = c_ax * c_half + it * _NS + s_ax
            r0 = pl.multiple_of(c * _R, _R)
            pltpu.sync_copy(d_hbm.at[c], idx_v)
            for kk in range(_P):
                pltpu.sync_copy(m_hbm.at[pl.ds(r0, _R), pl.ds(8 * kk, 8)],
                                upd_v)
                pltpu.sync_copy(upd_v, acc_sh.at[idx_v.at[kk]], add=True)
            return carry

        lax.fori_loop(0, iters, body, 0)
        plsc.subcore_barrier()
        pltpu.sync_copy(acc_sh.at[pl.ds(noff, per_tile_n), :],
                        out_hbm.at[c_ax, pl.ds(noff, per_tile_n), :])

    return k(dst8, em_p, zeros_n8)


def _sc_interleave(os):
    E5 = os[0].shape[0]
    C = E5 // _R
    G = 8
    n_grp = C // G
    n_lo = n_grp // _NW
    n_hi = n_grp - n_lo * _NW

    @functools.partial(
        pl.kernel,
        out_type=jax.ShapeDtypeStruct((E5 * _P, 1), jnp.float32),
        mesh=_mesh(),
        scratch_types=[pltpu.VMEM((G * _R, 1), jnp.float32)
                       for _ in range(_P)],
        compiler_params=_sc_params(),
    )
    def k(o0, o1, o2, o3, o4, out_hbm, b0, b1, b2, b3, b4):
        c_ax = lax.axis_index("c")
        s_ax = lax.axis_index("s")
        wid = s_ax * _NC + c_ax
        iters = n_lo + jnp.where(wid < n_hi, 1, 0)
        ins = [o0, o1, o2, o3, o4]
        bufs = [b0, b1, b2, b3, b4]

        def body(it, carry):
            g = it * _NW + wid
            r0 = pl.multiple_of(g * G * _R, G * _R)
            for kk in range(_P):
                pltpu.sync_copy(ins[kk].at[pl.ds(r0, G * _R), :], bufs[kk])
            for j in range(G):
                e0 = pl.multiple_of((r0 + j * _R) * _P, _R)
                for kk in range(_P):
                    pltpu.sync_copy(
                        bufs[kk].at[pl.ds(j * _R, _R), :],
                        out_hbm.at[pl.ds(e0 + kk * _R, _R), :])
            return carry

        lax.fori_loop(0, iters, body, 0)

    return k(*os)


def _kron5(w):
    return jnp.kron(jnp.eye(_P, dtype=jnp.float32), w)


def _r1_body(xea, w1, b1, w2, b2, w3, b3, out):
    h = jnp.dot(xea[...], w1[...], preferred_element_type=jnp.float32)
    h = jnp.maximum(h + b1[...], 0.0)
    h = jnp.dot(h, w2[...], preferred_element_type=jnp.float32)
    h = jnp.maximum(h + b2[...], 0.0)
    out[...] = jnp.dot(h, w3[...], preferred_element_type=jnp.float32) + b3[...]


def _r2_body(xij, em, w1, b1, w2, b2, w3, b3, o0, o1, o2, o3, o4):
    m = jnp.concatenate([xij[...], em[...]], axis=1)
    h = jnp.dot(m, w1[...], preferred_element_type=jnp.float32)
    h = jnp.maximum(h + b1[...], 0.0)
    h = jnp.dot(h, w2[...], preferred_element_type=jnp.float32)
    h = jnp.maximum(h + b2[...], 0.0)
    o = jnp.dot(h, w3[...], preferred_element_type=jnp.float32) + b3[...]
    o = jax.nn.sigmoid(o)
    outs = [o0, o1, o2, o3, o4]
    for kk in range(_P):
        outs[kk][...] = o[:, kk:kk + 1]


def _run_r1(xea, W1f, b1, W2, b2, W3p8, b3p8, *, bmr=2000):
    E5 = xea.shape[0]
    w1 = _kron5(W1f)
    w2 = _kron5(W2)
    w3 = _kron5(W3p8)
    b1b = jnp.tile(b1.reshape(1, -1), (1, _P))
    b2b = jnp.tile(b2.reshape(1, -1), (1, _P))
    b3b = jnp.tile(b3p8.reshape(1, -1) if b3p8.ndim == 1 else b3p8, (1, _P))
    eb = lambda n: pl.BlockSpec((bmr, n), lambda i: (i, 0))
    wb = lambda a: pl.BlockSpec(a.shape, lambda i: (0, 0))
    return pl.pallas_call(
        _r1_body,
        grid=(E5 // bmr,),
        in_specs=[eb(xea.shape[1]), wb(w1), wb(b1b), wb(w2), wb(b2b),
                  wb(w3), wb(b3b)],
        out_specs=eb(8 * _P),
        out_shape=jax.ShapeDtypeStruct((E5, 8 * _P), jnp.float32),
    )(xea, w1, b1b, w2, b2b, w3, b3b)


def _node_mlp_body(x, p0, p1, w1, b1, w2, b2, w3, b3, out):
    agg = p0[...] + p1[...]
    m = jnp.concatenate([x[...], agg[:, 0:4]], axis=1)
    h = jnp.dot(m, w1[...], preferred_element_type=jnp.float32) + b1[...]
    h = jnp.maximum(h, 0.0)
    h = jnp.dot(h, w2[...], preferred_element_type=jnp.float32) + b2[...]
    h = jnp.maximum(h, 0.0)
    out[...] = jnp.dot(h, w3[...], preferred_element_type=jnp.float32) + b3[...]


def _node_mlp(x, p0, p1, w1, b1, w2, b2, w3, b3, *, bn=2000):
    N = x.shape[0]
    eb = lambda n: pl.BlockSpec((bn, n), lambda i: (i, 0))
    wb = lambda a: pl.BlockSpec(a.shape, lambda i: (0, 0))
    return pl.pallas_call(
        _node_mlp_body,
        grid=(N // bn,),
        in_specs=[eb(3), eb(8), eb(8),
                  wb(w1), wb(b1), wb(w2), wb(b2), wb(w3), wb(b3)],
        out_specs=eb(8),
        out_shape=jax.ShapeDtypeStruct((N, 8), jnp.float32),
    )(x, p0, p1, w1, b1, w2, b2, w3, b3)


def _frame_w1(W1, xi_rows, xj_rows, ea_rows, fw):
    out = jnp.zeros((fw, W1.shape[1]), jnp.float32)
    out = out.at[0:3].set(W1[xi_rows])
    out = out.at[8:11].set(W1[xj_rows])
    if ea_rows is not None:
        out = out.at[16:20].set(W1[ea_rows])
    return out


def kernel(x, edge_index, edge_attr,
           R1_W1, R1_b1, R1_W2, R1_b2, R1_W3, R1_b3,
           O_W1, O_b1, O_W2, O_b2, O_W3, O_b3,
           R2_W1, R2_b1, R2_W2, R2_b2, R2_W3, R2_b3):
    N = x.shape[0]
    E = edge_index.shape[1]
    C = E // _CH

    xpad8 = jnp.concatenate([x, jnp.zeros((N, 5), x.dtype)], axis=1)
    idx3 = edge_index.reshape(2, C, _P, _R)
    pad = jnp.zeros((C, 8 - _P, _R), jnp.int32)
    dst8 = jnp.concatenate([idx3[1], pad], axis=1)
    src8 = jnp.concatenate([idx3[0], pad], axis=1)
    n_pad = ((N + _NS * 8 - 1) // (_NS * 8)) * (_NS * 8)
    zeros_n8 = jnp.zeros((n_pad, 8), jnp.float32)

    ea8 = jnp.concatenate(
        [edge_attr, jnp.zeros((E, 4), jnp.float32)], axis=1)
    xea = _sc_gather_packed(xpad8, dst8, src8, ea=ea8)
    w1f_r1 = _frame_w1(R1_W1, slice(0, 3), slice(3, 6), slice(6, 10), 24)
    w3p8_r1 = jnp.concatenate(
        [R1_W3, jnp.zeros((R1_W3.shape[0], 4), jnp.float32)], axis=1)
    b3p8_r1 = jnp.concatenate([R1_b3, jnp.zeros((4,), jnp.float32)])
    em_p = _run_r1(xea, w1f_r1, R1_b1, R1_W2, R1_b2, w3p8_r1, b3p8_r1)

    parts = _sc_segment_sum(dst8, em_p, zeros_n8)
    w3p8_o = jnp.concatenate(
        [O_W3, jnp.zeros((O_W3.shape[0], 5), jnp.float32)], axis=1)
    b3p8_o = jnp.concatenate([O_b3, jnp.zeros((5,), jnp.float32)])
    xt8 = _node_mlp(
        x, parts[0, :N], parts[1, :N],
        O_W1, O_b1.reshape(1, -1),
        O_W2, O_b2.reshape(1, -1),
        w3p8_o, b3p8_o.reshape(1, -1))

    xij2 = _sc_gather_packed(xt8, dst8, src8, ea=None)
    w1f_r2_x = _frame_w1(R2_W1, slice(0, 3), slice(3, 6), None, 16)
    w1f_r2_e = jnp.concatenate(
        [R2_W1[6:10], jnp.zeros((4, R2_W1.shape[1]), jnp.float32)], axis=0)
    os_ = _run_r2_full(xij2, em_p, w1f_r2_x, w1f_r2_e,
                       R2_b1, R2_W2, R2_b2, R2_W3, R2_b3)

    return _sc_interleave(os_)


def _run_r2_full(xij, em_p, w1x, w1e, b1, W2, b2, W3, b3, *, bmr=2000):
    E5 = xij.shape[0]
    w1 = jnp.concatenate([_kron5(w1x), _kron5(w1e)], axis=0)
    w2 = _kron5(W2)
    w3 = _kron5(W3)
    b1b = jnp.tile(b1.reshape(1, -1), (1, _P))
    b2b = jnp.tile(b2.reshape(1, -1), (1, _P))
    b3b = jnp.tile(b3.reshape(1, -1), (1, _P))
    eb = lambda n: pl.BlockSpec((bmr, n), lambda i: (i, 0))
    ob = pl.BlockSpec((bmr, 1), lambda i: (i, 0))
    wb = lambda a: pl.BlockSpec(a.shape, lambda i: (0, 0))
    return pl.pallas_call(
        _r2_body,
        grid=(E5 // bmr,),
        in_specs=[eb(xij.shape[1]), eb(em_p.shape[1]),
                  wb(w1), wb(b1b), wb(w2), wb(b2b), wb(w3), wb(b3b)],
        out_specs=[ob] * _P,
        out_shape=[jax.ShapeDtypeStruct((E5, 1), jnp.float32)] * _P,
    )(xij, em_p, w1, b1b, w2, b2b, w3, b3b)


# speedup vs baseline: 1.2033x; 1.0720x over previous
import functools

import jax
import jax.numpy as jnp
from jax import lax
from jax.experimental import pallas as pl
from jax.experimental.pallas import tpu as pltpu
from jax.experimental.pallas import tpu_sc as plsc

_NC = 2
_NS = 16
_NW = _NC * _NS

_P = 5
_R = 128
_CH = _P * _R


def _sc_params():
    return pltpu.CompilerParams(use_tc_tiling_on_sc=False)


def _mesh():
    return plsc.VectorSubcoreMesh(core_axis_name="c", subcore_axis_name="s")


def _sc_gather_packed(table8, dst8, src8, ea8=None, zpad=None):
    C = dst8.shape[0]
    E5 = C * _R
    FW = 24 if ea8 is not None else 16
    pad_w = 128 - FW * _P
    scratch = [
        pltpu.VMEM((8, _R), jnp.int32),
        pltpu.VMEM((8, _R), jnp.int32),
        pltpu.VMEM((_P * _R, 8), jnp.float32),
        pltpu.VMEM((_P * _R, 8), jnp.float32),
        pltpu.VMEM((_R, pad_w), jnp.float32),
        pltpu.SemaphoreType.DMA,
        pltpu.SemaphoreType.DMA,
    ]
    if ea8 is not None:
        scratch.append(pltpu.VMEM((_P * _R, 8), jnp.float32))
    n_lo = C // _NW
    n_hi_workers = C - n_lo * _NW

    @functools.partial(
        pl.kernel,
        out_type=jax.ShapeDtypeStruct((E5, 128), jnp.float32),
        mesh=_mesh(),
        scratch_types=scratch,
        compiler_params=_sc_params(),
    )
    def k(*refs):
        if ea8 is not None:
            (t_hbm, d_hbm, s_hbm, e_hbm, z_hbm, out_hbm,
             id_v, is_v, bd_v, bs_v, zb_v, sem_d, sem_s, be_v) = refs
        else:
            (t_hbm, d_hbm, s_hbm, z_hbm, out_hbm,
             id_v, is_v, bd_v, bs_v, zb_v, sem_d, sem_s) = refs
        c_ax = lax.axis_index("c")
        s_ax = lax.axis_index("s")
        wid = s_ax * _NC + c_ax
        pltpu.sync_copy(z_hbm, zb_v)
        iters = n_lo + jnp.where(wid < n_hi_workers, 1, 0)

        def body(it, carry):
            c = it * _NW + wid
            r0 = pl.multiple_of(c * _R, _R)
            pltpu.sync_copy(d_hbm.at[c], id_v)
            pltpu.sync_copy(s_hbm.at[c], is_v)
            cps = []
            for kk in range(_P):
                cps.append(pltpu.async_copy(
                    t_hbm.at[id_v.at[kk]],
                    bd_v.at[pl.ds(kk * _R, _R), :], sem_d))
                cps.append(pltpu.async_copy(
                    t_hbm.at[is_v.at[kk]],
                    bs_v.at[pl.ds(kk * _R, _R), :], sem_s))
            if ea8 is not None:
                e0 = pl.multiple_of(c * _CH, _CH)
                for kk in range(_P):
                    pltpu.sync_copy(e_hbm.at[pl.ds(e0 + kk * _R, _R), :],
                                    be_v.at[pl.ds(kk * _R, _R), :])
            for cp in cps:
                cp.wait()
            for kk in range(_P):
                pltpu.sync_copy(
                    bd_v.at[pl.ds(kk * _R, _R), :],
                    out_hbm.at[pl.ds(r0, _R), pl.ds(FW * kk, 8)])
                pltpu.sync_copy(
                    bs_v.at[pl.ds(kk * _R, _R), :],
                    out_hbm.at[pl.ds(r0, _R), pl.ds(FW * kk + 8, 8)])
            if ea8 is not None:
                for kk in range(_P):
                    pltpu.sync_copy(
                        be_v.at[pl.ds(kk * _R, _R), :],
                        out_hbm.at[pl.ds(r0, _R), pl.ds(FW * kk + 16, 8)])
            pltpu.sync_copy(zb_v,
                            out_hbm.at[pl.ds(r0, _R), pl.ds(FW * _P, pad_w)])
            return carry

        lax.fori_loop(0, iters, body, 0)

    if ea8 is not None:
        return k(table8, dst8, src8, ea8, zpad)
    return k(table8, dst8, src8, zpad)


def _sc_segment_sum(dst8, em_p, zeros_n8):
    NP = zeros_n8.shape[0]
    C = dst8.shape[0]
    per_tile_n = NP // _NS
    c_half = C // _NC

    @functools.partial(
        pl.kernel,
        out_type=jax.ShapeDtypeStruct((_NC, NP, 8), jnp.float32),
        mesh=_mesh(),
        scratch_types=[
            pltpu.VMEM((8, _R), jnp.int32),
            pltpu.VMEM((_R, 8), jnp.float32),
            pltpu.VMEM_SHARED((NP, 8), jnp.float32),
        ],
        compiler_params=_sc_params(),
    )
    def k(d_hbm, m_hbm, z_hbm, out_hbm, idx_v, upd_v, acc_sh):
        c_ax = lax.axis_index("c")
        s_ax = lax.axis_index("s")
        noff = pl.multiple_of(s_ax * per_tile_n, 8)
        pltpu.sync_copy(z_hbm.at[pl.ds(noff, per_tile_n), :],
                        acc_sh.at[pl.ds(noff, per_tile_n), :])
        plsc.subcore_barrier()
        n_lo = c_half // _NS
        n_hi = c_half - n_lo * _NS
        iters = n_lo + jnp.where(s_ax < n_hi, 1, 0)

        def body(it, carry):
            c = c_ax * c_half + it * _NS + s_ax
            r0 = pl.multiple_of(c * _R, _R)
            pltpu.sync_copy(d_hbm.at[c], idx_v)
            for kk in range(_P):
                pltpu.sync_copy(m_hbm.at[pl.ds(r0, _R), pl.ds(8 * kk, 8)],
                                upd_v)
                pltpu.sync_copy(upd_v, acc_sh.at[idx_v.at[kk]], add=True)
            return carry

        lax.fori_loop(0, iters, body, 0)
        plsc.subcore_barrier()
        pltpu.sync_copy(acc_sh.at[pl.ds(noff, per_tile_n), :],
                        out_hbm.at[c_ax, pl.ds(noff, per_tile_n), :])

    return k(dst8, em_p, zeros_n8)


def _sc_interleave(os):
    E5 = os[0].shape[0]
    C = E5 // _R
    G = 8
    n_grp = C // G
    n_lo = n_grp // _NW
    n_hi = n_grp - n_lo * _NW

    @functools.partial(
        pl.kernel,
        out_type=jax.ShapeDtypeStruct((E5 * _P, 1), jnp.float32),
        mesh=_mesh(),
        scratch_types=[pltpu.VMEM((G * _R, 1), jnp.float32)
                       for _ in range(_P)],
        compiler_params=_sc_params(),
    )
    def k(o0, o1, o2, o3, o4, out_hbm, b0, b1, b2, b3, b4):
        c_ax = lax.axis_index("c")
        s_ax = lax.axis_index("s")
        wid = s_ax * _NC + c_ax
        iters = n_lo + jnp.where(wid < n_hi, 1, 0)
        ins = [o0, o1, o2, o3, o4]
        bufs = [b0, b1, b2, b3, b4]

        def body(it, carry):
            g = it * _NW + wid
            r0 = pl.multiple_of(g * G * _R, G * _R)
            for kk in range(_P):
                pltpu.sync_copy(ins[kk].at[pl.ds(r0, G * _R), :], bufs[kk])
            for j in range(G):
                e0 = pl.multiple_of((r0 + j * _R) * _P, _R)
                for kk in range(_P):
                    pltpu.sync_copy(
                        bufs[kk].at[pl.ds(j * _R, _R), :],
                        out_hbm.at[pl.ds(e0 + kk * _R, _R), :])
            return carry

        lax.fori_loop(0, iters, body, 0)

    return k(*os)


def _kron5(w):
    return jnp.kron(jnp.eye(_P, dtype=jnp.float32), w)


def _r1_body(xea, w1, b1, w2, b2, w3, b3, out):
    h = jnp.dot(xea[...], w1[...], preferred_element_type=jnp.float32)
    h = jnp.maximum(h + b1[...], 0.0)
    h = jnp.dot(h, w2[...], preferred_element_type=jnp.float32)
    h = jnp.maximum(h + b2[...], 0.0)
    out[...] = jnp.dot(h, w3[...], preferred_element_type=jnp.float32) + b3[...]


def _r2_body(xij, em, w1, b1, w2, b2, w3, b3, o0, o1, o2, o3, o4):
    m = jnp.concatenate([xij[...], em[...][:, 0:8 * _P]], axis=1)
    h = jnp.dot(m, w1[...], preferred_element_type=jnp.float32)
    h = jnp.maximum(h + b1[...], 0.0)
    h = jnp.dot(h, w2[...], preferred_element_type=jnp.float32)
    h = jnp.maximum(h + b2[...], 0.0)
    o = jnp.dot(h, w3[...], preferred_element_type=jnp.float32) + b3[...]
    o = jax.nn.sigmoid(o)
    outs = [o0, o1, o2, o3, o4]
    for kk in range(_P):
        outs[kk][...] = o[:, kk:kk + 1]


def _run_r1(xea, W1f, b1, W2, b2, W3p8, b3p8, *, bmr=2000):
    E5 = xea.shape[0]
    w1 = jnp.concatenate(
        [_kron5(W1f), jnp.zeros((128 - _P * W1f.shape[0], _P * W1f.shape[1]),
                                jnp.float32)], axis=0)
    w2 = _kron5(W2)
    w3 = jnp.concatenate(
        [_kron5(W3p8),
         jnp.zeros((_P * W3p8.shape[0], 128 - _P * W3p8.shape[1]),
                   jnp.float32)], axis=1)
    b1b = jnp.tile(b1.reshape(1, -1), (1, _P))
    b2b = jnp.tile(b2.reshape(1, -1), (1, _P))
    b3t = jnp.tile(b3p8.reshape(1, -1), (1, _P))
    b3b = jnp.concatenate(
        [b3t, jnp.zeros((1, 128 - b3t.shape[1]), jnp.float32)], axis=1)
    eb = lambda n: pl.BlockSpec((bmr, n), lambda i: (i, 0))
    wb = lambda a: pl.BlockSpec(a.shape, lambda i: (0, 0))
    return pl.pallas_call(
        _r1_body,
        grid=(E5 // bmr,),
        in_specs=[eb(128), wb(w1), wb(b1b), wb(w2), wb(b2b),
                  wb(w3), wb(b3b)],
        out_specs=eb(128),
        out_shape=jax.ShapeDtypeStruct((E5, 128), jnp.float32),
    )(xea, w1, b1b, w2, b2b, w3, b3b)


def _node_mlp_body(x, p0, p1, w1, b1, w2, b2, w3, b3, out):
    agg = p0[...] + p1[...]
    m = jnp.concatenate([x[...], agg[:, 0:4]], axis=1)
    h = jnp.dot(m, w1[...], preferred_element_type=jnp.float32) + b1[...]
    h = jnp.maximum(h, 0.0)
    h = jnp.dot(h, w2[...], preferred_element_type=jnp.float32) + b2[...]
    h = jnp.maximum(h, 0.0)
    out[...] = jnp.dot(h, w3[...], preferred_element_type=jnp.float32) + b3[...]


def _node_mlp(x, p0, p1, w1, b1, w2, b2, w3, b3, *, bn=2000):
    N = x.shape[0]
    eb = lambda n: pl.BlockSpec((bn, n), lambda i: (i, 0))
    wb = lambda a: pl.BlockSpec(a.shape, lambda i: (0, 0))
    return pl.pallas_call(
        _node_mlp_body,
        grid=(N // bn,),
        in_specs=[eb(3), eb(8), eb(8),
                  wb(w1), wb(b1), wb(w2), wb(b2), wb(w3), wb(b3)],
        out_specs=eb(8),
        out_shape=jax.ShapeDtypeStruct((N, 8), jnp.float32),
    )(x, p0, p1, w1, b1, w2, b2, w3, b3)


def _frame_w1(W1, xi_rows, xj_rows, ea_rows, fw):
    out = jnp.zeros((fw, W1.shape[1]), jnp.float32)
    out = out.at[0:3].set(W1[xi_rows])
    out = out.at[8:11].set(W1[xj_rows])
    if ea_rows is not None:
        out = out.at[16:20].set(W1[ea_rows])
    return out


def kernel(x, edge_index, edge_attr,
           R1_W1, R1_b1, R1_W2, R1_b2, R1_W3, R1_b3,
           O_W1, O_b1, O_W2, O_b2, O_W3, O_b3,
           R2_W1, R2_b1, R2_W2, R2_b2, R2_W3, R2_b3):
    N = x.shape[0]
    E = edge_index.shape[1]
    C = E // _CH

    xpad8 = jnp.concatenate([x, jnp.zeros((N, 5), x.dtype)], axis=1)
    idx3 = edge_index.reshape(2, C, _P, _R)
    pad = jnp.zeros((C, 8 - _P, _R), jnp.int32)
    dst8 = jnp.concatenate([idx3[1], pad], axis=1)
    src8 = jnp.concatenate([idx3[0], pad], axis=1)
    n_pad = ((N + _NS * 8 - 1) // (_NS * 8)) * (_NS * 8)
    zeros_n8 = jnp.zeros((n_pad, 8), jnp.float32)

    ea8 = jnp.concatenate(
        [edge_attr, jnp.zeros((E, 4), jnp.float32)], axis=1)
    z8 = jnp.zeros((_R, 128 - 24 * _P), jnp.float32)
    z48 = jnp.zeros((_R, 128 - 16 * _P), jnp.float32)
    xea = _sc_gather_packed(xpad8, dst8, src8, ea8=ea8, zpad=z8)
    w1f_r1 = _frame_w1(R1_W1, slice(0, 3), slice(3, 6), slice(6, 10), 24)
    w3p8_r1 = jnp.concatenate(
        [R1_W3, jnp.zeros((R1_W3.shape[0], 4), jnp.float32)], axis=1)
    b3p8_r1 = jnp.concatenate([R1_b3, jnp.zeros((4,), jnp.float32)])
    em_p = _run_r1(xea, w1f_r1, R1_b1, R1_W2, R1_b2, w3p8_r1, b3p8_r1)

    parts = _sc_segment_sum(dst8, em_p, zeros_n8)
    w3p8_o = jnp.concatenate(
        [O_W3, jnp.zeros((O_W3.shape[0], 5), jnp.float32)], axis=1)
    b3p8_o = jnp.concatenate([O_b3, jnp.zeros((5,), jnp.float32)])
    xt8 = _node_mlp(
        x, parts[0, :N], parts[1, :N],
        O_W1, O_b1.reshape(1, -1),
        O_W2, O_b2.reshape(1, -1),
        w3p8_o, b3p8_o.reshape(1, -1))

    xij2 = _sc_gather_packed(xt8, dst8, src8, ea8=None, zpad=z48)
    w1f_r2_x = _frame_w1(R2_W1, slice(0, 3), slice(3, 6), None, 16)
    w1f_r2_e = jnp.concatenate(
        [R2_W1[6:10], jnp.zeros((4, R2_W1.shape[1]), jnp.float32)], axis=0)
    os_ = _run_r2_full(xij2, em_p, w1f_r2_x, w1f_r2_e,
                       R2_b1, R2_W2, R2_b2, R2_W3, R2_b3)

    return _sc_interleave(os_)


def _run_r2_full(xij, em_p, w1x, w1e, b1, W2, b2, W3, b3, *, bmr=2000):
    E5 = xij.shape[0]
    w1 = jnp.concatenate(
        [_kron5(w1x),
         jnp.zeros((128 - _P * w1x.shape[0], _P * w1x.shape[1]), jnp.float32),
         _kron5(w1e)], axis=0)
    w2 = _kron5(W2)
    w3 = _kron5(W3)
    b1b = jnp.tile(b1.reshape(1, -1), (1, _P))
    b2b = jnp.tile(b2.reshape(1, -1), (1, _P))
    b3b = jnp.tile(b3.reshape(1, -1), (1, _P))
    eb = lambda n: pl.BlockSpec((bmr, n), lambda i: (i, 0))
    ob = pl.BlockSpec((bmr, 1), lambda i: (i, 0))
    wb = lambda a: pl.BlockSpec(a.shape, lambda i: (0, 0))
    return pl.pallas_call(
        _r2_body,
        grid=(E5 // bmr,),
        in_specs=[eb(128), eb(128),
                  wb(w1), wb(b1b), wb(w2), wb(b2b), wb(w3), wb(b3b)],
        out_specs=[ob] * _P,
        out_shape=[jax.ShapeDtypeStruct((E5, 1), jnp.float32)] * _P,
    )(xij, em_p, w1, b1b, w2, b2b, w3, b3b)


# speedup vs baseline: 1.4228x; 1.1825x over previous
import functools

import jax
import jax.numpy as jnp
from jax import lax
from jax.experimental import pallas as pl
from jax.experimental.pallas import tpu as pltpu
from jax.experimental.pallas import tpu_sc as plsc

_NC = 2
_NS = 16
_NW = _NC * _NS

_P = 5
_R = 128
_CH = _P * _R


def _sc_params():
    return pltpu.CompilerParams(use_tc_tiling_on_sc=False)


def _mesh():
    return plsc.VectorSubcoreMesh(core_axis_name="c", subcore_axis_name="s")


def _sc_gather_packed(table8, dst8, src8, ea8=None, zpad=None):
    C = dst8.shape[0]
    E5 = C * _R
    FW = 24 if ea8 is not None else 16
    pad_w = 128 - FW * _P
    scratch = [
        pltpu.VMEM((8, _R), jnp.int32),
        pltpu.VMEM((8, _R), jnp.int32),
        pltpu.VMEM((_P * _R, 8), jnp.float32),
        pltpu.VMEM((_P * _R, 8), jnp.float32),
        pltpu.VMEM((_R, pad_w), jnp.float32),
        pltpu.SemaphoreType.DMA,
        pltpu.SemaphoreType.DMA,
    ]
    if ea8 is not None:
        scratch.append(pltpu.VMEM((_P * _R, 8), jnp.float32))
    n_lo = C // _NW
    n_hi_workers = C - n_lo * _NW

    @functools.partial(
        pl.kernel,
        out_type=jax.ShapeDtypeStruct((E5, 128), jnp.float32),
        mesh=_mesh(),
        scratch_types=scratch,
        compiler_params=_sc_params(),
    )
    def k(*refs):
        if ea8 is not None:
            (t_hbm, d_hbm, s_hbm, e_hbm, z_hbm, out_hbm,
             id_v, is_v, bd_v, bs_v, zb_v, sem_d, sem_s, be_v) = refs
        else:
            (t_hbm, d_hbm, s_hbm, z_hbm, out_hbm,
             id_v, is_v, bd_v, bs_v, zb_v, sem_d, sem_s) = refs
        c_ax = lax.axis_index("c")
        s_ax = lax.axis_index("s")
        wid = s_ax * _NC + c_ax
        pltpu.sync_copy(z_hbm, zb_v)
        iters = n_lo + jnp.where(wid < n_hi_workers, 1, 0)

        def body(it, carry):
            c = it * _NW + wid
            r0 = pl.multiple_of(c * _R, _R)
            pltpu.sync_copy(d_hbm.at[c], id_v)
            pltpu.sync_copy(s_hbm.at[c], is_v)
            cps = []
            for kk in range(_P):
                cps.append(pltpu.async_copy(
                    t_hbm.at[id_v.at[kk]],
                    bd_v.at[pl.ds(kk * _R, _R), :], sem_d))
                cps.append(pltpu.async_copy(
                    t_hbm.at[is_v.at[kk]],
                    bs_v.at[pl.ds(kk * _R, _R), :], sem_s))
            if ea8 is not None:
                e0 = pl.multiple_of(c * _CH, _CH)
                for kk in range(_P):
                    pltpu.sync_copy(e_hbm.at[pl.ds(e0 + kk * _R, _R), :],
                                    be_v.at[pl.ds(kk * _R, _R), :])
            for cp in cps:
                cp.wait()
            for kk in range(_P):
                pltpu.sync_copy(
                    bd_v.at[pl.ds(kk * _R, _R), :],
                    out_hbm.at[pl.ds(r0, _R), pl.ds(FW * kk, 8)])
                pltpu.sync_copy(
                    bs_v.at[pl.ds(kk * _R, _R), :],
                    out_hbm.at[pl.ds(r0, _R), pl.ds(FW * kk + 8, 8)])
            if ea8 is not None:
                for kk in range(_P):
                    pltpu.sync_copy(
                        be_v.at[pl.ds(kk * _R, _R), :],
                        out_hbm.at[pl.ds(r0, _R), pl.ds(FW * kk + 16, 8)])
            pltpu.sync_copy(zb_v,
                            out_hbm.at[pl.ds(r0, _R), pl.ds(FW * _P, pad_w)])
            return carry

        lax.fori_loop(0, iters, body, 0)

    if ea8 is not None:
        return k(table8, dst8, src8, ea8, zpad)
    return k(table8, dst8, src8, zpad)


def _sc_segment_sum(dst8, em_p, zeros_n8):
    NP = zeros_n8.shape[0]
    C = dst8.shape[0]
    per_tile_n = NP // _NS
    c_half = C // _NC

    @functools.partial(
        pl.kernel,
        out_type=jax.ShapeDtypeStruct((_NC, NP, 8), jnp.float32),
        mesh=_mesh(),
        scratch_types=[
            pltpu.VMEM((8, _R), jnp.int32),
            pltpu.VMEM((_R, 8), jnp.float32),
            pltpu.VMEM_SHARED((NP, 8), jnp.float32),
        ],
        compiler_params=_sc_params(),
    )
    def k(d_hbm, m_hbm, z_hbm, out_hbm, idx_v, upd_v, acc_sh):
        c_ax = lax.axis_index("c")
        s_ax = lax.axis_index("s")
        noff = pl.multiple_of(s_ax * per_tile_n, 8)
        pltpu.sync_copy(z_hbm.at[pl.ds(noff, per_tile_n), :],
                        acc_sh.at[pl.ds(noff, per_tile_n), :])
        plsc.subcore_barrier()
        n_lo = c_half // _NS
        n_hi = c_half - n_lo * _NS
        iters = n_lo + jnp.where(s_ax < n_hi, 1, 0)

        def body(it, carry):
            c = c_ax * c_half + it * _NS + s_ax
            r0 = pl.multiple_of(c * _R, _R)
            pltpu.sync_copy(d_hbm.at[c], idx_v)
            for kk in range(_P):
                pltpu.sync_copy(m_hbm.at[pl.ds(r0, _R), pl.ds(8 * kk, 8)],
                                upd_v)
                pltpu.sync_copy(upd_v, acc_sh.at[idx_v.at[kk]], add=True)
            return carry

        lax.fori_loop(0, iters, body, 0)
        plsc.subcore_barrier()
        pltpu.sync_copy(acc_sh.at[pl.ds(noff, per_tile_n), :],
                        out_hbm.at[c_ax, pl.ds(noff, per_tile_n), :])

    return k(dst8, em_p, zeros_n8)


def _sc_interleave(os):
    E5 = os[0].shape[0]
    C = E5 // _R
    G = 8
    n_grp = C // G
    n_lo = n_grp // _NW
    n_hi = n_grp - n_lo * _NW

    @functools.partial(
        pl.kernel,
        out_type=jax.ShapeDtypeStruct((E5 * _P, 1), jnp.float32),
        mesh=_mesh(),
        scratch_types=[pltpu.VMEM((G * _R, 1), jnp.float32)
                       for _ in range(_P)],
        compiler_params=_sc_params(),
    )
    def k(o0, o1, o2, o3, o4, out_hbm, b0, b1, b2, b3, b4):
        c_ax = lax.axis_index("c")
        s_ax = lax.axis_index("s")
        wid = s_ax * _NC + c_ax
        iters = n_lo + jnp.where(wid < n_hi, 1, 0)
        ins = [o0, o1, o2, o3, o4]
        bufs = [b0, b1, b2, b3, b4]

        def body(it, carry):
            g = it * _NW + wid
            r0 = pl.multiple_of(g * G * _R, G * _R)
            for kk in range(_P):
                pltpu.sync_copy(ins[kk].at[pl.ds(r0, G * _R), :], bufs[kk])
            for j in range(G):
                e0 = pl.multiple_of((r0 + j * _R) * _P, _R)
                for kk in range(_P):
                    pltpu.sync_copy(
                        bufs[kk].at[pl.ds(j * _R, _R), :],
                        out_hbm.at[pl.ds(e0 + kk * _R, _R), :])
            return carry

        lax.fori_loop(0, iters, body, 0)

    return k(*os)


def _kron5(w):
    return jnp.kron(jnp.eye(_P, dtype=jnp.float32), w)


def _r1_body(xea, w1, b1, w2, b2, w3, b3, out):
    h = jnp.dot(xea[...], w1[...], preferred_element_type=jnp.float32)
    h = jnp.maximum(h + b1[...], 0.0)
    h = jnp.dot(h, w2[...], preferred_element_type=jnp.float32)
    h = jnp.maximum(h + b2[...], 0.0)
    out[...] = jnp.dot(h, w3[...], preferred_element_type=jnp.float32) + b3[...]


def _r2_body(xij, em, w1, b1, w2, b2, w3, b3, o0, o1, o2, o3, o4):
    m = jnp.concatenate([xij[...], em[...][:, 0:8 * _P]], axis=1)
    h = jnp.dot(m, w1[...], preferred_element_type=jnp.float32)
    h = jnp.maximum(h + b1[...], 0.0)
    h = jnp.dot(h, w2[...], preferred_element_type=jnp.float32)
    h = jnp.maximum(h + b2[...], 0.0)
    o = jnp.dot(h, w3[...], preferred_element_type=jnp.float32) + b3[...]
    o = jax.nn.sigmoid(o)
    outs = [o0, o1, o2, o3, o4]
    for kk in range(_P):
        outs[kk][...] = o[:, kk:kk + 1]


def _run_r1(xea, W1f, b1, W2, b2, W3p8, b3p8, *, bmr=2000):
    E5 = xea.shape[0]
    w1 = jnp.concatenate(
        [_kron5(W1f), jnp.zeros((128 - _P * W1f.shape[0], _P * W1f.shape[1]),
                                jnp.float32)], axis=0)
    w2 = _kron5(W2)
    w3 = jnp.concatenate(
        [_kron5(W3p8),
         jnp.zeros((_P * W3p8.shape[0], 128 - _P * W3p8.shape[1]),
                   jnp.float32)], axis=1)
    b1b = jnp.tile(b1.reshape(1, -1), (1, _P))
    b2b = jnp.tile(b2.reshape(1, -1), (1, _P))
    b3t = jnp.tile(b3p8.reshape(1, -1), (1, _P))
    b3b = jnp.concatenate(
        [b3t, jnp.zeros((1, 128 - b3t.shape[1]), jnp.float32)], axis=1)
    eb = lambda n: pl.BlockSpec((bmr, n), lambda i: (i, 0))
    wb = lambda a: pl.BlockSpec(a.shape, lambda i: (0, 0))
    return pl.pallas_call(
        _r1_body,
        grid=(E5 // bmr,),
        in_specs=[eb(128), wb(w1), wb(b1b), wb(w2), wb(b2b),
                  wb(w3), wb(b3b)],
        out_specs=eb(128),
        out_shape=jax.ShapeDtypeStruct((E5, 128), jnp.float32),
    )(xea, w1, b1b, w2, b2b, w3, b3b)


def _widen_body(ea, out):
    v = ea[...]
    out[...] = jnp.concatenate([v, jnp.zeros_like(v)], axis=1)


def _widen_ea(ea, *, bm=8192):
    E = ea.shape[0]
    return pl.pallas_call(
        _widen_body,
        grid=(E // bm,),
        in_specs=[pl.BlockSpec((bm, 4), lambda i: (i, 0))],
        out_specs=pl.BlockSpec((bm, 8), lambda i: (i, 0)),
        out_shape=jax.ShapeDtypeStruct((E, 8), jnp.float32),
    )(ea)


def _node_mlp_body(x, p0, p1, w1, b1, w2, b2, w3, b3, out):
    agg = p0[...] + p1[...]
    m = jnp.concatenate([x[...], agg[:, 0:4]], axis=1)
    h = jnp.dot(m, w1[...], preferred_element_type=jnp.float32) + b1[...]
    h = jnp.maximum(h, 0.0)
    h = jnp.dot(h, w2[...], preferred_element_type=jnp.float32) + b2[...]
    h = jnp.maximum(h, 0.0)
    out[...] = jnp.dot(h, w3[...], preferred_element_type=jnp.float32) + b3[...]


def _node_mlp(x, p0, p1, w1, b1, w2, b2, w3, b3, *, bn=2000):
    N = x.shape[0]
    eb = lambda n: pl.BlockSpec((bn, n), lambda i: (i, 0))
    wb = lambda a: pl.BlockSpec(a.shape, lambda i: (0, 0))
    return pl.pallas_call(
        _node_mlp_body,
        grid=(N // bn,),
        in_specs=[eb(3), eb(8), eb(8),
                  wb(w1), wb(b1), wb(w2), wb(b2), wb(w3), wb(b3)],
        out_specs=eb(8),
        out_shape=jax.ShapeDtypeStruct((N, 8), jnp.float32),
    )(x, p0, p1, w1, b1, w2, b2, w3, b3)


def _frame_w1(W1, xi_rows, xj_rows, ea_rows, fw):
    out = jnp.zeros((fw, W1.shape[1]), jnp.float32)
    out = out.at[0:3].set(W1[xi_rows])
    out = out.at[8:11].set(W1[xj_rows])
    if ea_rows is not None:
        out = out.at[16:20].set(W1[ea_rows])
    return out


def kernel(x, edge_index, edge_attr,
           R1_W1, R1_b1, R1_W2, R1_b2, R1_W3, R1_b3,
           O_W1, O_b1, O_W2, O_b2, O_W3, O_b3,
           R2_W1, R2_b1, R2_W2, R2_b2, R2_W3, R2_b3):
    N = x.shape[0]
    E = edge_index.shape[1]
    C = E // _CH

    xpad8 = jnp.concatenate([x, jnp.zeros((N, 5), x.dtype)], axis=1)
    idx3 = edge_index.reshape(2, C, _P, _R)
    pad = jnp.zeros((C, 8 - _P, _R), jnp.int32)
    dst8 = jnp.concatenate([idx3[1], pad], axis=1)
    src8 = jnp.concatenate([idx3[0], pad], axis=1)
    n_pad = ((N + _NS * 8 - 1) // (_NS * 8)) * (_NS * 8)
    zeros_n8 = jnp.zeros((n_pad, 8), jnp.float32)

    ea8 = _widen_ea(edge_attr)
    z8 = jnp.zeros((_R, 128 - 24 * _P), jnp.float32)
    z48 = jnp.zeros((_R, 128 - 16 * _P), jnp.float32)
    xea = _sc_gather_packed(xpad8, dst8, src8, ea8=ea8, zpad=z8)
    w1f_r1 = _frame_w1(R1_W1, slice(0, 3), slice(3, 6), slice(6, 10), 24)
    w3p8_r1 = jnp.concatenate(
        [R1_W3, jnp.zeros((R1_W3.shape[0], 4), jnp.float32)], axis=1)
    b3p8_r1 = jnp.concatenate([R1_b3, jnp.zeros((4,), jnp.float32)])
    em_p = _run_r1(xea, w1f_r1, R1_b1, R1_W2, R1_b2, w3p8_r1, b3p8_r1)

    parts = _sc_segment_sum(dst8, em_p, zeros_n8)
    w3p8_o = jnp.concatenate(
        [O_W3, jnp.zeros((O_W3.shape[0], 5), jnp.float32)], axis=1)
    b3p8_o = jnp.concatenate([O_b3, jnp.zeros((5,), jnp.float32)])
    xt8 = _node_mlp(
        x, parts[0, :N], parts[1, :N],
        O_W1, O_b1.reshape(1, -1),
        O_W2, O_b2.reshape(1, -1),
        w3p8_o, b3p8_o.reshape(1, -1))

    xij2 = _sc_gather_packed(xt8, dst8, src8, ea8=None, zpad=z48)
    w1f_r2_x = _frame_w1(R2_W1, slice(0, 3), slice(3, 6), None, 16)
    w1f_r2_e = jnp.concatenate(
        [R2_W1[6:10], jnp.zeros((4, R2_W1.shape[1]), jnp.float32)], axis=0)
    os_ = _run_r2_full(xij2, em_p, w1f_r2_x, w1f_r2_e,
                       R2_b1, R2_W2, R2_b2, R2_W3, R2_b3)

    return _sc_interleave(os_)


def _run_r2_full(xij, em_p, w1x, w1e, b1, W2, b2, W3, b3, *, bmr=2000):
    E5 = xij.shape[0]
    w1 = jnp.concatenate(
        [_kron5(w1x),
         jnp.zeros((128 - _P * w1x.shape[0], _P * w1x.shape[1]), jnp.float32),
         _kron5(w1e)], axis=0)
    w2 = _kron5(W2)
    w3 = _kron5(W3)
    b1b = jnp.tile(b1.reshape(1, -1), (1, _P))
    b2b = jnp.tile(b2.reshape(1, -1), (1, _P))
    b3b = jnp.tile(b3.reshape(1, -1), (1, _P))
    eb = lambda n: pl.BlockSpec((bmr, n), lambda i: (i, 0))
    ob = pl.BlockSpec((bmr, 1), lambda i: (i, 0))
    wb = lambda a: pl.BlockSpec(a.shape, lambda i: (0, 0))
    return pl.pallas_call(
        _r2_body,
        grid=(E5 // bmr,),
        in_specs=[eb(128), eb(128),
                  wb(w1), wb(b1b), wb(w2), wb(b2b), wb(w3), wb(b3b)],
        out_specs=[ob] * _P,
        out_shape=[jax.ShapeDtypeStruct((E5, 1), jnp.float32)] * _P,
    )(xij, em_p, w1, b1b, w2, b2b, w3, b3b)
